# baseline (speedup 1.0000x reference)
import functools

import jax
import jax.numpy as jnp
from jax import lax
from jax.experimental import pallas as pl
from jax.experimental.pallas import tpu as pltpu
from jax.experimental.pallas import tpu_sc as plsc

_BATCH = 4096
_SEQ = 200
_D = 64
_NC = 2
_NS = 16
_NW = _NC * _NS
_BPW = _BATCH // _NW


def _sc_body(xT, tok, pos_hbm, out, idx_v, rows_v, pos_v, sem):
    wid = lax.axis_index("s") * _NC + lax.axis_index("c")
    b0 = wid * _BPW
    pltpu.sync_copy(pos_hbm, pos_v)

    def unit(s, carry):
        pltpu.sync_copy(xT.at[s, wid], idx_v)
        pltpu.async_copy(tok.at[idx_v], rows_v, sem).wait()
        pv0 = pos_v[s, pl.ds(0, 16)]
        pv1 = pos_v[s, pl.ds(16, 16)]
        pv2 = pos_v[s, pl.ds(32, 16)]
        pv3 = pos_v[s, pl.ds(48, 16)]

        def row(r, c2):
            rows_v[r, pl.ds(0, 16)] = rows_v[r, pl.ds(0, 16)] + pv0
            rows_v[r, pl.ds(16, 16)] = rows_v[r, pl.ds(16, 16)] + pv1
            rows_v[r, pl.ds(32, 16)] = rows_v[r, pl.ds(32, 16)] + pv2
            rows_v[r, pl.ds(48, 16)] = rows_v[r, pl.ds(48, 16)] + pv3
            return c2

        lax.fori_loop(0, _BPW, row, 0)
        pltpu.sync_copy(rows_v, out.at[pl.ds(b0, _BPW), s])
        return carry

    lax.fori_loop(0, _SEQ, unit, 0)


@jax.jit
def kernel(x, token_table, pos_table):
    xT = jnp.asarray(x, jnp.int32).T.reshape(_SEQ, _NW, _BPW)
    mesh = plsc.VectorSubcoreMesh(core_axis_name="c", subcore_axis_name="s")
    f = functools.partial(
        pl.kernel,
        mesh=mesh,
        out_type=jax.ShapeDtypeStruct((_BATCH, _SEQ, _D), jnp.float32),
        scratch_types=[
            pltpu.VMEM((_BPW,), jnp.int32),
            pltpu.VMEM((_BPW, _D), jnp.float32),
            pltpu.VMEM((_SEQ, _D), jnp.float32),
            pltpu.SemaphoreType.DMA,
        ],
        compiler_params=pltpu.CompilerParams(use_tc_tiling_on_sc=False),
    )(_sc_body)
    return f(xT, token_table, pos_table.astype(jnp.float32))


# speedup vs baseline: 1.2734x; 1.2734x over previous
import functools

import jax
import jax.numpy as jnp
from jax import lax
from jax.experimental import pallas as pl
from jax.experimental.pallas import tpu as pltpu
from jax.experimental.pallas import tpu_sc as plsc

_BATCH = 4096
_SEQ = 200
_D = 64
_NC = 2
_NS = 16
_NW = _NC * _NS
_BPW = _BATCH // _NW
_NB = 4


def _sc_body(xW, tok, pos_hbm, out, idx_v, rows_v, pos_v, *sems):
    gsem = sems[:_NB]
    wsem = sems[_NB:]
    wid = lax.axis_index("s") * _NC + lax.axis_index("c")
    b0 = wid * _BPW
    pltpu.sync_copy(xW.at[wid], idx_v)
    pltpu.sync_copy(pos_hbm, pos_v)

    def fire_gather(s, b):
        pltpu.make_async_copy(tok.at[idx_v.at[s]], rows_v.at[b], gsem[b]).start()

    def wait_gather(s, b):
        pltpu.make_async_copy(tok.at[idx_v.at[s]], rows_v.at[b], gsem[b]).wait()

    def fire_write(s, b):
        pltpu.make_async_copy(
            rows_v.at[b], out.at[pl.ds(b0, _BPW), s], wsem[b]
        ).start()

    def wait_write(s, b):
        pltpu.make_async_copy(
            rows_v.at[b], out.at[pl.ds(b0, _BPW), s], wsem[b]
        ).wait()

    for b in range(_NB - 1):
        fire_gather(b, b)

    def step(t, carry):
        for b in range(_NB):
            s = t * _NB + b
            wait_gather(s, b)
            rb = rows_v.at[b]
            pv0 = pos_v[s, pl.ds(0, 16)]
            pv1 = pos_v[s, pl.ds(16, 16)]
            pv2 = pos_v[s, pl.ds(32, 16)]
            pv3 = pos_v[s, pl.ds(48, 16)]

            def row(k, c2, rb=rb, pv0=pv0, pv1=pv1, pv2=pv2, pv3=pv3):
                for u in range(4):
                    r = k * 4 + u
                    rb[r, pl.ds(0, 16)] = rb[r, pl.ds(0, 16)] + pv0
                    rb[r, pl.ds(16, 16)] = rb[r, pl.ds(16, 16)] + pv1
                    rb[r, pl.ds(32, 16)] = rb[r, pl.ds(32, 16)] + pv2
                    rb[r, pl.ds(48, 16)] = rb[r, pl.ds(48, 16)] + pv3
                return c2

            lax.fori_loop(0, _BPW // 4, row, 0)
            fire_write(s, b)
            bp = (b - 1) % _NB
            sp = s - 1

            @pl.when(s >= 1)
            def _():
                wait_write(sp, bp)

            s2 = s + _NB - 1

            @pl.when(s2 < _SEQ)
            def _():
                fire_gather(s2, bp)
        return carry

    lax.fori_loop(0, _SEQ // _NB, step, 0)
    wait_write(_SEQ - 1, _NB - 1)


@jax.jit
def kernel(x, token_table, pos_table):
    xW = jnp.asarray(x, jnp.int32).T.reshape(_SEQ, _NW, _BPW).transpose(1, 0, 2)
    mesh = plsc.VectorSubcoreMesh(core_axis_name="c", subcore_axis_name="s")
    f = functools.partial(
        pl.kernel,
        mesh=mesh,
        out_type=jax.ShapeDtypeStruct((_BATCH, _SEQ, _D), jnp.float32),
        scratch_types=[
            pltpu.VMEM((_SEQ, _BPW), jnp.int32),
            pltpu.VMEM((_NB, _BPW, _D), jnp.float32),
            pltpu.VMEM((_SEQ, _D), jnp.float32),
        ]
        + [pltpu.SemaphoreType.DMA] * (2 * _NB),
        compiler_params=pltpu.CompilerParams(use_tc_tiling_on_sc=False),
    )(_sc_body)
    return f(xW, token_table, pos_table.astype(jnp.float32))
